# baseline (speedup 1.0000x reference)
import functools

import jax
import jax.numpy as jnp
from jax import lax
from jax.experimental import pallas as pl
from jax.experimental.pallas import tpu as pltpu
from jax.experimental.pallas import tpu_sc as plsc

_NUM_ENT = 100000
_HIDDEN = 16
_BATCH = 1024
_PACK = 128 // _HIDDEN

_info = plsc.get_sparse_core_info()
_NC, _NS = _info.num_cores, _info.num_subcores
_NW = _NC * _NS
_BPW = _BATCH // _NW

_sc_mesh = plsc.VectorSubcoreMesh(core_axis_name="c", subcore_axis_name="s")


@functools.partial(
    pl.kernel,
    mesh=_sc_mesh,
    out_type=jax.ShapeDtypeStruct((_BATCH, _HIDDEN), jnp.float32),
    scratch_types=[
        pltpu.VMEM((_BPW,), jnp.int32),
        pltpu.VMEM((_BPW,), jnp.int32),
        pltpu.VMEM((_BPW, 128), jnp.float32),
        pltpu.VMEM((_BPW, _HIDDEN), jnp.float32),
        pltpu.SemaphoreType.DMA,
    ],
    compiler_params=pltpu.CompilerParams(needs_layout_passes=False),
)
def _sc_gather(table_hbm, idx_hbm, out_hbm, idx_v, row_v, rows_v, emb_v, sem):
    wid = lax.axis_index("s") * _NC + lax.axis_index("c")
    base = wid * _BPW
    pltpu.sync_copy(idx_hbm.at[pl.ds(base, _BPW)], idx_v)
    for g in range(_BPW // 16):
        v = idx_v[pl.ds(g * 16, 16)]
        row_v[pl.ds(g * 16, 16)] = lax.shift_right_logical(v, 3)
    pltpu.async_copy(table_hbm.at[row_v], rows_v, sem).wait()
    lanes = lax.iota(jnp.int32, 16)
    for g in range(_BPW // 16):
        v = idx_v[pl.ds(g * 16, 16)]
        col0 = (v & (_PACK - 1)) * _HIDDEN
        rowsel = g * 16 + lanes
        for k in range(_HIDDEN):
            vals = plsc.load_gather(rows_v, [rowsel, col0 + k])
            plsc.store_scatter(
                emb_v, [rowsel, jnp.full((16,), k, jnp.int32)], vals)
    pltpu.sync_copy(emb_v, out_hbm.at[pl.ds(base, _BPW)])


_BN = 2048
_NT = (_NUM_ENT + _BN - 1) // _BN


def _mm_body(emb_ref, w_ref, b_ref, out_ref):
    acc = lax.dot_general(
        emb_ref[...], w_ref[...],
        (((1,), (1,)), ((), ())),
        preferred_element_type=jnp.float32,
    )
    out_ref[...] = acc + b_ref[...]


def kernel(batch_data, ent_emb, W, b):
    idx = batch_data.reshape(_BATCH).astype(jnp.int32)
    table2 = ent_emb.reshape(_NUM_ENT // _PACK, 128)
    emb = _sc_gather(table2, idx)
    b2 = b.reshape(1, _NUM_ENT)
    out = pl.pallas_call(
        _mm_body,
        grid=(_NT,),
        in_specs=[
            pl.BlockSpec((_BATCH, _HIDDEN), lambda i: (0, 0)),
            pl.BlockSpec((_BN, _HIDDEN), lambda i: (i, 0)),
            pl.BlockSpec((1, _BN), lambda i: (0, i)),
        ],
        out_specs=pl.BlockSpec((_BATCH, _BN), lambda i: (0, i)),
        out_shape=jax.ShapeDtypeStruct((_BATCH, _NUM_ENT), jnp.float32),
    )(emb, W, b2)
    return out


# speedup vs baseline: 2.7128x; 2.7128x over previous
import functools

import jax
import jax.numpy as jnp
from jax import lax
from jax.experimental import pallas as pl
from jax.experimental.pallas import tpu as pltpu
from jax.experimental.pallas import tpu_sc as plsc

_NUM_ENT = 100000
_HIDDEN = 16
_BATCH = 1024

_info = plsc.get_sparse_core_info()
_NC, _NS = _info.num_cores, _info.num_subcores
_NW = _NC * _NS
_BPW = _BATCH // _NW
_NPOS = _BPW * _HIDDEN
_CHUNK = 128

_sc_mesh = plsc.VectorSubcoreMesh(core_axis_name="c", subcore_axis_name="s")


@functools.partial(
    pl.kernel,
    mesh=_sc_mesh,
    out_type=jax.ShapeDtypeStruct((_BATCH * _HIDDEN,), jnp.float32),
    scratch_types=[
        pltpu.VMEM((_BPW,), jnp.int32),
        pltpu.VMEM((_NPOS,), jnp.int32),
        pltpu.VMEM((_NPOS,), jnp.float32),
        pltpu.SemaphoreType.DMA,
    ],
    compiler_params=pltpu.CompilerParams(needs_layout_passes=False),
)
def _sc_gather(table_hbm, idx_hbm, out_hbm, idx_v, pos_v, val_v, sem):
    wid = lax.axis_index("s") * _NC + lax.axis_index("c")
    base = wid * _BPW
    pltpu.sync_copy(idx_hbm.at[pl.ds(base, _BPW)], idx_v)
    lanes = lax.iota(jnp.int32, 16)
    for g in range(_BPW // 16):
        v = idx_v[pl.ds(g * 16, 16)]
        dst0 = g * 256 + lanes * 16
        for k in range(_HIDDEN):
            plsc.store_scatter(pos_v, [dst0 + k], v + k * _NUM_ENT)
    copies = [
        pltpu.async_copy(
            table_hbm.at[pos_v.at[pl.ds(c * _CHUNK, _CHUNK)]],
            val_v.at[pl.ds(c * _CHUNK, _CHUNK)],
            sem,
        )
        for c in range(_NPOS // _CHUNK)
    ]
    for cp in copies:
        cp.wait()
    pltpu.sync_copy(val_v, out_hbm.at[pl.ds(base * _HIDDEN, _NPOS)])


_BN = 2048
_NT = (_NUM_ENT + _BN - 1) // _BN


def _mm_body(wt_ref, emb_ref, b_ref, out_ref):
    acc = lax.dot_general(
        wt_ref[...], emb_ref[...],
        (((0,), (1,)), ((), ())),
        preferred_element_type=jnp.float32,
    )
    out_ref[...] = acc + b_ref[...]


def kernel(batch_data, ent_emb, W, b):
    idx = batch_data.reshape(_BATCH).astype(jnp.int32)
    table_flat = ent_emb.T.reshape(_NUM_ENT * _HIDDEN)
    emb = _sc_gather(table_flat, idx).reshape(_BATCH, _HIDDEN)
    wt = W.T
    bcol = b.reshape(_NUM_ENT, 1)
    out_t = pl.pallas_call(
        _mm_body,
        grid=(_NT,),
        in_specs=[
            pl.BlockSpec((_HIDDEN, _BN), lambda i: (0, i)),
            pl.BlockSpec((_BATCH, _HIDDEN), lambda i: (0, 0)),
            pl.BlockSpec((_BN, 1), lambda i: (i, 0)),
        ],
        out_specs=pl.BlockSpec((_BN, _BATCH), lambda i: (i, 0)),
        out_shape=jax.ShapeDtypeStruct((_NUM_ENT, _BATCH), jnp.float32),
    )(wt, emb, bcol)
    return out_t.T


# speedup vs baseline: 2.9124x; 1.0736x over previous
import functools

import jax
import jax.numpy as jnp
from jax import lax
from jax.experimental import pallas as pl
from jax.experimental.pallas import tpu as pltpu
from jax.experimental.pallas import tpu_sc as plsc

_NUM_ENT = 100000
_HIDDEN = 16
_BATCH = 1024

_info = plsc.get_sparse_core_info()
_NC, _NS = _info.num_cores, _info.num_subcores
_NW = _NC * _NS
_BPW = _BATCH // _NW
_NPOS = _BPW * _HIDDEN
_CHUNK = 128

_sc_mesh = plsc.VectorSubcoreMesh(core_axis_name="c", subcore_axis_name="s")


@functools.partial(
    pl.kernel,
    mesh=_sc_mesh,
    out_type=jax.ShapeDtypeStruct((_BATCH * _HIDDEN,), jnp.float32),
    scratch_types=[
        pltpu.VMEM((_BPW,), jnp.int32),
        pltpu.VMEM((_NPOS,), jnp.int32),
        pltpu.VMEM((_NPOS,), jnp.float32),
        pltpu.SemaphoreType.DMA,
    ],
    compiler_params=pltpu.CompilerParams(needs_layout_passes=False),
)
def _sc_gather(table_hbm, idx_hbm, out_hbm, idx_v, pos_v, val_v, sem):
    wid = lax.axis_index("s") * _NC + lax.axis_index("c")
    base = wid * _BPW
    pltpu.sync_copy(idx_hbm.at[pl.ds(base, _BPW)], idx_v)
    lanes = lax.iota(jnp.int32, 16)
    for g in range(_BPW // 16):
        v = idx_v[pl.ds(g * 16, 16)] * _HIDDEN
        dst0 = g * 256 + lanes * 16
        for k in range(_HIDDEN):
            plsc.store_scatter(pos_v, [dst0 + k], v + k)
    copies = [
        pltpu.async_copy(
            table_hbm.at[pos_v.at[pl.ds(c * _CHUNK, _CHUNK)]],
            val_v.at[pl.ds(c * _CHUNK, _CHUNK)],
            sem,
        )
        for c in range(_NPOS // _CHUNK)
    ]
    for cp in copies:
        cp.wait()
    pltpu.sync_copy(val_v, out_hbm.at[pl.ds(base * _HIDDEN, _NPOS)])


_KA = _HIDDEN + 1
_BN = 2048
_NT = (_NUM_ENT + _BN - 1) // _BN


def _mm_body(wa_ref, ea_ref, out_ref):
    out_ref[...] = lax.dot_general(
        wa_ref[...], ea_ref[...],
        (((0,), (1,)), ((), ())),
        preferred_element_type=jnp.float32,
    )


def kernel(batch_data, ent_emb, W, b):
    idx = batch_data.reshape(_BATCH).astype(jnp.int32)
    table_flat = ent_emb.reshape(_NUM_ENT * _HIDDEN)
    emb = _sc_gather(table_flat, idx).reshape(_BATCH, _HIDDEN)
    ea = jnp.concatenate(
        [emb, jnp.ones((_BATCH, 1), jnp.float32)], axis=1)
    wa = jnp.concatenate([W.T, b.reshape(1, _NUM_ENT)], axis=0)
    out_t = pl.pallas_call(
        _mm_body,
        grid=(_NT,),
        in_specs=[
            pl.BlockSpec((_KA, _BN), lambda i: (0, i)),
            pl.BlockSpec((_BATCH, _KA), lambda i: (0, 0)),
        ],
        out_specs=pl.BlockSpec((_BN, _BATCH), lambda i: (i, 0)),
        out_shape=jax.ShapeDtypeStruct((_NUM_ENT, _BATCH), jnp.float32),
    )(wa, ea)
    return out_t.T
